# Initial kernel scaffold; baseline (speedup 1.0000x reference)
import jax
import jax.numpy as jnp
from jax.experimental import pallas as pl


def kernel(input_len, table):
    raise NotImplementedError("write your pallas kernel here")


# speedup vs baseline: 1.0973x; 1.0973x over previous
import functools

import jax
import jax.numpy as jnp
from jax import lax
from jax.experimental import pallas as pl
from jax.experimental.pallas import tpu as pltpu
from jax.experimental.pallas import tpu_sc as plsc

B = 16
MAX_LEN = 2048
D = 1024

NC = 2
NS = 16
NW = NC * NS
ROWS_PER_W = B * MAX_LEN // NW
CHUNK = 32
NCHUNK = ROWS_PER_W // CHUNK

_mesh = plsc.VectorSubcoreMesh(core_axis_name="c", subcore_axis_name="s")


@functools.partial(
    pl.kernel,
    mesh=_mesh,
    out_type=jax.ShapeDtypeStruct((B * MAX_LEN, D), jnp.float32),
    scratch_types=[
        pltpu.VMEM((B,), jnp.int32),
        pltpu.VMEM((CHUNK,), jnp.int32),
        pltpu.VMEM((CHUNK, D), jnp.float32),
        pltpu.SemaphoreType.DMA,
    ],
)
def _pos_enc_sc(len_hbm, table_hbm, out_hbm, len_v, idx_v, buf_v, sem):
    wid = lax.axis_index("s") * NC + lax.axis_index("c")
    b = wid // 2
    half = wid % 2
    i0 = half * ROWS_PER_W

    pltpu.sync_copy(len_hbm, len_v)
    lens = len_v[...]
    lane = lax.iota(jnp.int32, 16)
    bvec = jnp.broadcast_to(b, (16,)).astype(jnp.int32)
    len_b = lax.gather(
        lens,
        bvec[:, None],
        dimension_numbers=lax.GatherDimensionNumbers(
            offset_dims=(), collapsed_slice_dims=(0,), start_index_map=(0,)),
        slice_sizes=(1,),
        mode=lax.GatherScatterMode.PROMISE_IN_BOUNDS,
    )

    row_out0 = wid * ROWS_PER_W

    def chunk_body(c, _):
        base_i = i0 + c * CHUNK
        for j in range(CHUNK // 16):
            rows = base_i + j * 16 + lane
            idx = jnp.where(rows < len_b, rows + 1, 0)
            idx_v[pl.ds(j * 16, 16)] = idx
        pltpu.async_copy(table_hbm.at[idx_v], buf_v, sem).wait()
        pltpu.sync_copy(buf_v, out_hbm.at[pl.ds(row_out0 + c * CHUNK, CHUNK)])
        return _

    lax.fori_loop(0, NCHUNK, chunk_body, None)


def kernel(input_len, table):
    out = _pos_enc_sc(input_len, table)
    return out.reshape(B, MAX_LEN, D)
